# Initial kernel scaffold; baseline (speedup 1.0000x reference)
import jax
import jax.numpy as jnp
from jax.experimental import pallas as pl


def kernel(x, visible, vis_mask, memory_pos, memory_neg):
    raise NotImplementedError("write your pallas kernel here")


# speedup vs baseline: 2.4843x; 2.4843x over previous
import jax
import jax.numpy as jnp
from jax.experimental import pallas as pl
from jax.experimental.pallas import tpu as pltpu

NUM_NOISE = 16

BM = 512
BN = 2048


def _sim_kernel(a_ref, nz_ref, b_ref, sim_ref, nsim_ref):
    dims = (((1,), (1,)), ((), ()))
    sim_ref[...] = jax.lax.dot_general(
        a_ref[...], b_ref[...], dims, preferred_element_type=jnp.float32
    )

    @pl.when(pl.program_id(1) == 0)
    def _():
        nsim_ref[...] = jax.lax.dot_general(
            nz_ref[...], b_ref[...], dims, preferred_element_type=jnp.float32
        )


def kernel(x, visible, vis_mask, memory_pos, memory_neg):
    b, k, c = x.shape
    num_pos = k - NUM_NOISE
    t_sel = x[:, :num_pos, :].reshape(b * num_pos, c)
    noise = x[:, num_pos:, :].reshape(b * NUM_NOISE, c)
    memory = jnp.concatenate((memory_pos, memory_neg), axis=0)

    m = b * num_pos
    n = memory.shape[0]
    nrows = b * NUM_NOISE

    sim, nsim = pl.pallas_call(
        _sim_kernel,
        grid=(n // BN, m // BM),
        in_specs=[
            pl.BlockSpec((BM, c), lambda j, i: (i, 0)),
            pl.BlockSpec((nrows, c), lambda j, i: (0, 0)),
            pl.BlockSpec((BN, c), lambda j, i: (j, 0)),
        ],
        out_specs=[
            pl.BlockSpec((BM, BN), lambda j, i: (i, j)),
            pl.BlockSpec((nrows, BN), lambda j, i: (0, j)),
        ],
        out_shape=[
            jax.ShapeDtypeStruct((m, n), jnp.float32),
            jax.ShapeDtypeStruct((nrows, n), jnp.float32),
        ],
        compiler_params=pltpu.CompilerParams(
            dimension_semantics=("parallel", "parallel")
        ),
    )(t_sel, noise, memory)

    noise_similarity = nsim[:, : memory_pos.shape[0]].reshape(b, NUM_NOISE, -1)
    return sim, noise_similarity


# speedup vs baseline: 2.8800x; 1.1593x over previous
import jax
import jax.numpy as jnp
from jax.experimental import pallas as pl
from jax.experimental.pallas import tpu as pltpu

NUM_NOISE = 16

BM = 1024
BN = 2048


def _sim_kernel(a_ref, nz_ref, b_ref, sim_ref, nsim_ref):
    dims = (((1,), (1,)), ((), ()))
    sim_ref[...] = jax.lax.dot_general(
        a_ref[...], b_ref[...], dims, preferred_element_type=jnp.float32
    )

    @pl.when(pl.program_id(1) == 0)
    def _():
        nsim_ref[...] = jax.lax.dot_general(
            nz_ref[...], b_ref[...], dims, preferred_element_type=jnp.float32
        )


def kernel(x, visible, vis_mask, memory_pos, memory_neg):
    b, k, c = x.shape
    num_pos = k - NUM_NOISE
    t_sel = x[:, :num_pos, :].reshape(b * num_pos, c)
    noise = x[:, num_pos:, :].reshape(b * NUM_NOISE, c)
    memory = jnp.concatenate((memory_pos, memory_neg), axis=0)

    m = b * num_pos
    n = memory.shape[0]
    nrows = b * NUM_NOISE

    sim, nsim = pl.pallas_call(
        _sim_kernel,
        grid=(n // BN, m // BM),
        in_specs=[
            pl.BlockSpec((BM, c), lambda j, i: (i, 0)),
            pl.BlockSpec((nrows, c), lambda j, i: (0, 0)),
            pl.BlockSpec((BN, c), lambda j, i: (j, 0)),
        ],
        out_specs=[
            pl.BlockSpec((BM, BN), lambda j, i: (i, j)),
            pl.BlockSpec((nrows, BN), lambda j, i: (0, j)),
        ],
        out_shape=[
            jax.ShapeDtypeStruct((m, n), jnp.float32),
            jax.ShapeDtypeStruct((nrows, n), jnp.float32),
        ],
        compiler_params=pltpu.CompilerParams(
            dimension_semantics=("parallel", "parallel")
        ),
    )(t_sel, noise, memory)

    noise_similarity = nsim[:, : memory_pos.shape[0]].reshape(b, NUM_NOISE, -1)
    return sim, noise_similarity


# speedup vs baseline: 2.9595x; 1.0276x over previous
import jax
import jax.numpy as jnp
from jax.experimental import pallas as pl
from jax.experimental.pallas import tpu as pltpu

NUM_NOISE = 16

BM = 1024
BN = 4096


def _sim_kernel(a_ref, nz_ref, b_ref, sim_ref, nsim_ref):
    dims = (((1,), (1,)), ((), ()))
    sim_ref[...] = jax.lax.dot_general(
        a_ref[...], b_ref[...], dims, preferred_element_type=jnp.float32
    )

    @pl.when(pl.program_id(1) == 0)
    def _():
        nsim_ref[...] = jax.lax.dot_general(
            nz_ref[...], b_ref[...], dims, preferred_element_type=jnp.float32
        )


def kernel(x, visible, vis_mask, memory_pos, memory_neg):
    b, k, c = x.shape
    num_pos = k - NUM_NOISE
    t_sel = x[:, :num_pos, :].reshape(b * num_pos, c)
    noise = x[:, num_pos:, :].reshape(b * NUM_NOISE, c)
    memory = jnp.concatenate((memory_pos, memory_neg), axis=0)

    m = b * num_pos
    n = memory.shape[0]
    nrows = b * NUM_NOISE

    sim, nsim = pl.pallas_call(
        _sim_kernel,
        grid=(n // BN, m // BM),
        in_specs=[
            pl.BlockSpec((BM, c), lambda j, i: (i, 0)),
            pl.BlockSpec((nrows, c), lambda j, i: (0, 0)),
            pl.BlockSpec((BN, c), lambda j, i: (j, 0)),
        ],
        out_specs=[
            pl.BlockSpec((BM, BN), lambda j, i: (i, j)),
            pl.BlockSpec((nrows, BN), lambda j, i: (0, j)),
        ],
        out_shape=[
            jax.ShapeDtypeStruct((m, n), jnp.float32),
            jax.ShapeDtypeStruct((nrows, n), jnp.float32),
        ],
        compiler_params=pltpu.CompilerParams(
            dimension_semantics=("parallel", "parallel")
        ),
    )(t_sel, noise, memory)

    noise_similarity = nsim[:, : memory_pos.shape[0]].reshape(b, NUM_NOISE, -1)
    return sim, noise_similarity
